# baseline (speedup 1.0000x reference)
import functools

import jax
import jax.numpy as jnp
from jax.experimental import pallas as pl

N = 8192
K = 8192
D = 32
BLK = 512
CHUNK = 2048


def _vq_kernel(x_ref, w_ref, q_ref, idx_ref, loss_ref):
    x = x_ref[...]
    w = w_ref[...]
    x2 = jnp.sum(x * x, axis=1, keepdims=True)
    w2 = jnp.sum(w * w, axis=1)
    mm = jax.lax.dot_general(
        x.astype(jnp.bfloat16), w.astype(jnp.bfloat16),
        (((1,), (1,)), ((), ())),
        preferred_element_type=jnp.float32)
    negd = -((x2 - 2.0 * mm) + w2[None, :])

    acc_v = jnp.full((x.shape[0], 1), -jnp.inf, jnp.float32)
    acc_i = jnp.zeros((x.shape[0], 1), jnp.int32)
    for c in range(0, K, CHUNK):
        t = negd[:, c:c + CHUNK]
        tm = jnp.max(t, axis=1, keepdims=True)
        ti = jnp.argmax(t, axis=1)[:, None] + c
        gt = tm > acc_v
        eq = (tm == acc_v) & (ti < acc_i)
        acc_i = jnp.where(gt | eq, ti, acc_i)
        acc_v = jnp.where(gt, tm, acc_v).astype(jnp.bfloat16).astype(jnp.float32)
    idx = acc_i[:, 0]

    onehot = (jax.lax.broadcasted_iota(jnp.int32, negd.shape, 1)
              == acc_i).astype(jnp.float32)
    q = jax.lax.dot_general(
        onehot, w, (((1,), (0,)), ((), ())),
        precision=jax.lax.Precision.HIGHEST,
        preferred_element_type=jnp.float32)
    q_ref[...] = q
    idx_ref[...] = idx
    r = q - x
    part = jnp.sum(r * r).reshape(1, 1)

    @pl.when(pl.program_id(0) == 0)
    def _():
        loss_ref[...] = jnp.zeros_like(loss_ref)

    loss_ref[...] += part


@functools.partial(jax.jit, static_argnames=("interpret",))
def kernel(inputs, w, interpret=False):
    x = inputs.reshape(-1, D)
    n = x.shape[0]
    grid = (n // BLK,)
    q, idx, loss_sum = pl.pallas_call(
        _vq_kernel,
        grid=grid,
        in_specs=[
            pl.BlockSpec((BLK, D), lambda i: (i, 0)),
            pl.BlockSpec((K, D), lambda i: (0, 0)),
        ],
        out_specs=[
            pl.BlockSpec((BLK, D), lambda i: (i, 0)),
            pl.BlockSpec((BLK,), lambda i: (i,)),
            pl.BlockSpec((1, 1), lambda i: (0, 0)),
        ],
        out_shape=[
            jax.ShapeDtypeStruct((n, D), jnp.float32),
            jax.ShapeDtypeStruct((n,), jnp.int32),
            jax.ShapeDtypeStruct((1, 1), jnp.float32),
        ],
        interpret=interpret,
    )(x, w)
    beta = 0.25
    loss = beta * (loss_sum[0, 0] / inputs.size)
    quantized_st = q.reshape(inputs.shape)
    return (quantized_st, loss, idx, w.T)


# speedup vs baseline: 1.6254x; 1.6254x over previous
import functools

import jax
import jax.numpy as jnp
from jax.experimental import pallas as pl

N = 8192
K = 8192
D = 32
BLK = 512
CHUNK = 2048


def _vq_kernel(x_ref, w_ref, q_ref, idx_ref, loss_ref):
    x = x_ref[...]
    w = w_ref[...]
    x2 = jnp.sum(x * x, axis=1, keepdims=True)
    w2 = jnp.sum(w * w, axis=1)
    mm = jax.lax.dot_general(
        x.astype(jnp.bfloat16), w.astype(jnp.bfloat16),
        (((1,), (1,)), ((), ())),
        preferred_element_type=jnp.float32)
    negd = -((x2 - 2.0 * mm) + w2[None, :])

    acc_v = jnp.full((x.shape[0], 1), -jnp.inf, jnp.float32)
    acc_i = jnp.zeros((x.shape[0], 1), jnp.int32)
    for c in range(0, K, CHUNK):
        t = negd[:, c:c + CHUNK]
        tm = jnp.max(t, axis=1, keepdims=True)
        ti = jnp.argmax(t, axis=1)[:, None] + c
        gt = tm > acc_v
        eq = (tm == acc_v) & (ti < acc_i)
        acc_i = jnp.where(gt | eq, ti, acc_i)
        acc_v = jnp.where(gt, tm, acc_v).astype(jnp.bfloat16).astype(jnp.float32)
    idx = acc_i[:, 0]

    onehot = (jax.lax.broadcasted_iota(jnp.int32, negd.shape, 1)
              == acc_i).astype(jnp.bfloat16)
    w_hi = w.astype(jnp.bfloat16)
    w_lo = (w - w_hi.astype(jnp.float32)).astype(jnp.bfloat16)
    dims = (((1,), (0,)), ((), ()))
    q = (jax.lax.dot_general(onehot, w_hi, dims,
                             preferred_element_type=jnp.float32)
         + jax.lax.dot_general(onehot, w_lo, dims,
                               preferred_element_type=jnp.float32))
    q_ref[...] = q
    idx_ref[...] = idx
    r = q - x
    part = jnp.sum(r * r).reshape(1, 1)

    @pl.when(pl.program_id(0) == 0)
    def _():
        loss_ref[...] = jnp.zeros_like(loss_ref)

    loss_ref[...] += part


@functools.partial(jax.jit, static_argnames=("interpret",))
def kernel(inputs, w, interpret=False):
    x = inputs.reshape(-1, D)
    n = x.shape[0]
    grid = (n // BLK,)
    q, idx, loss_sum = pl.pallas_call(
        _vq_kernel,
        grid=grid,
        in_specs=[
            pl.BlockSpec((BLK, D), lambda i: (i, 0)),
            pl.BlockSpec((K, D), lambda i: (0, 0)),
        ],
        out_specs=[
            pl.BlockSpec((BLK, D), lambda i: (i, 0)),
            pl.BlockSpec((BLK,), lambda i: (i,)),
            pl.BlockSpec((1, 1), lambda i: (0, 0)),
        ],
        out_shape=[
            jax.ShapeDtypeStruct((n, D), jnp.float32),
            jax.ShapeDtypeStruct((n,), jnp.int32),
            jax.ShapeDtypeStruct((1, 1), jnp.float32),
        ],
        interpret=interpret,
    )(x, w)
    beta = 0.25
    loss = beta * (loss_sum[0, 0] / inputs.size)
    quantized_st = q.reshape(inputs.shape)
    return (quantized_st, loss, idx, w.T)


# speedup vs baseline: 1.7481x; 1.0755x over previous
import functools

import jax
import jax.numpy as jnp
from jax.experimental import pallas as pl

N = 8192
K = 8192
D = 32
BLK = 1024
CHUNK = 2048


def _vq_kernel(x_ref, xb_ref, wb_ref, whi_ref, wlo_ref, w2_ref,
               q_ref, idx_ref, loss_ref):
    x = x_ref[...]
    xb = xb_ref[...]
    wb = wb_ref[...]
    x2 = jnp.sum(x * x, axis=1, keepdims=True)
    w2 = w2_ref[...]
    mm = jax.lax.dot_general(
        xb, wb, (((1,), (1,)), ((), ())),
        preferred_element_type=jnp.float32)
    d = (x2 - 2.0 * mm) + w2

    acc_v = jnp.full((x.shape[0], 1), jnp.inf, jnp.float32)
    acc_i = jnp.zeros((x.shape[0], 1), jnp.int32)
    for c in range(0, K, CHUNK):
        t = d[:, c:c + CHUNK]
        tm = jnp.min(t, axis=1, keepdims=True)
        ti = jnp.argmin(t, axis=1)[:, None] + c
        lt = tm < acc_v
        eq = (tm == acc_v) & (ti < acc_i)
        acc_i = jnp.where(lt | eq, ti, acc_i)
        acc_v = jnp.where(lt, tm, acc_v).astype(jnp.bfloat16).astype(jnp.float32)
    idx = acc_i[:, 0]

    onehot = (jax.lax.broadcasted_iota(jnp.int32, d.shape, 1)
              == acc_i).astype(jnp.bfloat16)
    dims = (((1,), (0,)), ((), ()))
    q = (jax.lax.dot_general(onehot, whi_ref[...], dims,
                             preferred_element_type=jnp.float32)
         + jax.lax.dot_general(onehot, wlo_ref[...], dims,
                               preferred_element_type=jnp.float32))
    q_ref[...] = q
    idx_ref[...] = idx
    r = q - x
    part = jnp.sum(r * r).reshape(1, 1)

    @pl.when(pl.program_id(0) == 0)
    def _():
        loss_ref[...] = jnp.zeros_like(loss_ref)

    loss_ref[...] += part


@functools.partial(jax.jit, static_argnames=("interpret",))
def kernel(inputs, w, interpret=False):
    x = inputs.reshape(-1, D)
    n = x.shape[0]
    xb = x.astype(jnp.bfloat16)
    wb = w.astype(jnp.bfloat16)
    w_hi = wb
    w_lo = (w - w_hi.astype(jnp.float32)).astype(jnp.bfloat16)
    w2 = jnp.sum(w * w, axis=1)[None, :]
    grid = (n // BLK,)
    q, idx, loss_sum = pl.pallas_call(
        _vq_kernel,
        grid=grid,
        in_specs=[
            pl.BlockSpec((BLK, D), lambda i: (i, 0)),
            pl.BlockSpec((BLK, D), lambda i: (i, 0)),
            pl.BlockSpec((K, D), lambda i: (0, 0)),
            pl.BlockSpec((K, D), lambda i: (0, 0)),
            pl.BlockSpec((K, D), lambda i: (0, 0)),
            pl.BlockSpec((1, K), lambda i: (0, 0)),
        ],
        out_specs=[
            pl.BlockSpec((BLK, D), lambda i: (i, 0)),
            pl.BlockSpec((BLK,), lambda i: (i,)),
            pl.BlockSpec((1, 1), lambda i: (0, 0)),
        ],
        out_shape=[
            jax.ShapeDtypeStruct((n, D), jnp.float32),
            jax.ShapeDtypeStruct((n,), jnp.int32),
            jax.ShapeDtypeStruct((1, 1), jnp.float32),
        ],
        interpret=interpret,
    )(x, xb, wb, w_hi, w_lo, w2)
    beta = 0.25
    loss = beta * (loss_sum[0, 0] / inputs.size)
    quantized_st = q.reshape(inputs.shape)
    return (quantized_st, loss, idx, w.T)
